# Initial kernel scaffold; baseline (speedup 1.0000x reference)
import jax
import jax.numpy as jnp
from jax.experimental import pallas as pl


def kernel(x, edge_index, W1_rel, b1, W1_root, W2_rel, b2, W2_root, W3_rel, b3, W3_root):
    raise NotImplementedError("write your pallas kernel here")


# speedup vs baseline: 4.7284x; 4.7284x over previous
import functools

import jax
import jax.numpy as jnp
from jax import lax
from jax.experimental import pallas as pl
from jax.experimental.pallas import tpu as pltpu
from jax.experimental.pallas import tpu_sc as plsc

_NC = 2
_NS = 16
_C = 128
_W = 128


def _seg_sum_sc(table, src3, dst3, r_rows, split_features):
    R = r_rows
    RPT = R // _NS
    NCH = src3.shape[-2]
    zeros = jnp.zeros((R, _W), jnp.float32)

    mesh = plsc.VectorSubcoreMesh(core_axis_name="c", subcore_axis_name="s")

    @functools.partial(
        pl.kernel,
        mesh=mesh,
        out_type=jax.ShapeDtypeStruct((_NC, R, _W), jnp.float32),
        scratch_types=[
            pltpu.VMEM_SHARED((R, _W), jnp.float32),
            pltpu.VMEM((_C,), jnp.int32),
            pltpu.VMEM((_C,), jnp.int32),
            pltpu.VMEM((_C, _W), jnp.float32),
            pltpu.SemaphoreType.DMA,
        ],
    )
    def seg_sum(tbl_hbm, src_hbm, dst_hbm, z_hbm, out_hbm,
                acc, src_cur, dst_cur, rows, sem):
        c = lax.axis_index("c")
        s = lax.axis_index("s")
        r0 = s * RPT
        pltpu.sync_copy(z_hbm.at[pl.ds(r0, RPT)], acc.at[pl.ds(r0, RPT)])
        plsc.subcore_barrier()

        def chunk(j, carry):
            if split_features:
                pltpu.sync_copy(src_hbm.at[c, s, j], src_cur)
                pltpu.sync_copy(dst_hbm.at[s, j], dst_cur)
            else:
                w = c * _NS + s
                pltpu.sync_copy(src_hbm.at[w, j], src_cur)
                pltpu.sync_copy(dst_hbm.at[w, j], dst_cur)
            pltpu.async_copy(tbl_hbm.at[src_cur], rows, sem).wait()
            pltpu.sync_copy(rows, acc.at[dst_cur], add=True)
            return carry

        lax.fori_loop(0, NCH, chunk, 0)
        plsc.subcore_barrier()
        pltpu.sync_copy(acc.at[pl.ds(r0, RPT)], out_hbm.at[c, pl.ds(r0, RPT)])

    return seg_sum(table, src3, dst3, zeros)


def _dotT(a, b):
    return lax.dot_general(a, b, (((1,), (1,)), ((), ())),
                           preferred_element_type=jnp.float32)


def _layer1_tc(accp, x, W1_rel, b1, W1_root, W2a, W2b, BR=1024):
    N, DIN = x.shape
    H1 = W1_rel.shape[0]
    NB = -(-N // BR)

    def body(acc_ref, x_ref, wr_ref, b_ref, wt_ref, w2a_ref, w2b_ref,
             h1_ref, y2_ref):
        a = acc_ref[...]
        h1 = _dotT(a[0] + a[1], wr_ref[...]) + b_ref[...]
        h1 = jnp.maximum(h1 + _dotT(x_ref[...], wt_ref[...]), 0.0)
        h1_ref[...] = h1
        y2_ref[0] = _dotT(h1, w2a_ref[...])
        y2_ref[1] = _dotT(h1, w2b_ref[...])

    return pl.pallas_call(
        body,
        grid=(NB,),
        in_specs=[
            pl.BlockSpec((2, BR, DIN), lambda i: (0, i, 0)),
            pl.BlockSpec((BR, DIN), lambda i: (i, 0)),
            pl.BlockSpec((H1, DIN), lambda i: (0, 0)),
            pl.BlockSpec((1, H1), lambda i: (0, 0)),
            pl.BlockSpec((H1, DIN), lambda i: (0, 0)),
            pl.BlockSpec((_W, H1), lambda i: (0, 0)),
            pl.BlockSpec((_W, H1), lambda i: (0, 0)),
        ],
        out_specs=[
            pl.BlockSpec((BR, H1), lambda i: (i, 0)),
            pl.BlockSpec((2, BR, _W), lambda i: (0, i, 0)),
        ],
        out_shape=[
            jax.ShapeDtypeStruct((N, H1), jnp.float32),
            jax.ShapeDtypeStruct((2, N, _W), jnp.float32),
        ],
    )(accp, x, W1_rel, b1.reshape(1, -1), W1_root, W2a, W2b)


def _layer2_tc(accf, h1, W2_root, b2, W3p, BR=1024):
    N, H1 = h1.shape
    H2 = W2_root.shape[0]
    NB = -(-N // BR)

    def body(acc_ref, h1_ref, wt_ref, b_ref, w3_ref, h2_ref, y3_ref):
        a = acc_ref[...]
        agg = jnp.concatenate([a[0], a[1, :, :H2 - _W]], axis=1)
        h2 = jnp.maximum(
            agg + b_ref[...] + _dotT(h1_ref[...], wt_ref[...]), 0.0)
        h2_ref[...] = h2
        y3_ref[...] = _dotT(h2, w3_ref[...])

    return pl.pallas_call(
        body,
        grid=(NB,),
        in_specs=[
            pl.BlockSpec((2, BR, _W), lambda i: (0, i, 0)),
            pl.BlockSpec((BR, H1), lambda i: (i, 0)),
            pl.BlockSpec((H2, H1), lambda i: (0, 0)),
            pl.BlockSpec((1, H2), lambda i: (0, 0)),
            pl.BlockSpec((_W, H2), lambda i: (0, 0)),
        ],
        out_specs=[
            pl.BlockSpec((BR, H2), lambda i: (i, 0)),
            pl.BlockSpec((BR, _W), lambda i: (i, 0)),
        ],
        out_shape=[
            jax.ShapeDtypeStruct((N, H2), jnp.float32),
            jax.ShapeDtypeStruct((N, _W), jnp.float32),
        ],
    )(accf, h1, W2_root, b2.reshape(1, -1), W3p)


def _layer3_tc(accp, h2, W3_root, b3, OUT, BR=1024):
    N, H2 = h2.shape
    NB = -(-N // BR)

    def body(acc_ref, h2_ref, wt_ref, b_ref, out_ref):
        a = acc_ref[...]
        agg = a[0, :, :OUT] + a[1, :, :OUT]
        out_ref[...] = jnp.maximum(
            agg + b_ref[...] + _dotT(h2_ref[...], wt_ref[...]), 0.0)

    return pl.pallas_call(
        body,
        grid=(NB,),
        in_specs=[
            pl.BlockSpec((2, BR, _W), lambda i: (0, i, 0)),
            pl.BlockSpec((BR, H2), lambda i: (i, 0)),
            pl.BlockSpec((OUT, H2), lambda i: (0, 0)),
            pl.BlockSpec((1, OUT), lambda i: (0, 0)),
        ],
        out_specs=pl.BlockSpec((BR, OUT), lambda i: (i, 0)),
        out_shape=jax.ShapeDtypeStruct((N, OUT), jnp.float32),
    )(accp, h2, W3_root, b3.reshape(1, -1))


def _pad_ids(ids, nslabs, nch, fill):
    pad = nslabs * nch * _C - ids.shape[0]
    return jnp.concatenate(
        [ids, jnp.full((pad,), fill, jnp.int32)]).reshape(nslabs, nch, _C)


def kernel(x, edge_index, W1_rel, b1, W1_root, W2_rel, b2, W2_root,
           W3_rel, b3, W3_root):
    N, DIN = x.shape
    E = edge_index.shape[1]
    OUT = W3_rel.shape[0]
    H2 = W2_rel.shape[0]
    R = -(-(N + 1) // 128) * 128

    src = edge_index[0]
    dst = edge_index[1]
    NCH32 = -(-E // (2 * _NS * _C))
    src32 = _pad_ids(src, 2 * _NS, NCH32, 0)
    dst32 = _pad_ids(dst, 2 * _NS, NCH32, N)
    NCH16 = -(-E // (_NS * _C))
    src16 = _pad_ids(src, _NS, NCH16, 0)
    src16 = jnp.stack([src16, src16 + N])
    dst16 = _pad_ids(dst, _NS, NCH16, N)

    acc1 = _seg_sum_sc(x, src32, dst32, R, split_features=False)
    W2a = W2_rel[:_W]
    W2b = jnp.pad(W2_rel[_W:], ((0, 2 * _W - H2), (0, 0)))
    h1, y2t = _layer1_tc(acc1, x, W1_rel, b1, W1_root, W2a, W2b)

    acc2 = _seg_sum_sc(y2t.reshape(2 * N, _W), src16, dst16, R,
                       split_features=True)
    W3p = jnp.pad(W3_rel, ((0, _W - OUT), (0, 0)))
    h2, y3p = _layer2_tc(acc2, h1, W2_root, b2, W3p)

    acc3 = _seg_sum_sc(y3p, src32, dst32, R, split_features=False)
    return _layer3_tc(acc3, h2, W3_root, b3, OUT)
